# Initial kernel scaffold; baseline (speedup 1.0000x reference)
import jax
import jax.numpy as jnp
from jax.experimental import pallas as pl


def kernel(x, edge_index, W, b):
    raise NotImplementedError("write your pallas kernel here")


# speedup vs baseline: 9.9616x; 9.9616x over previous
import jax
import jax.numpy as jnp
from jax import lax
from jax.experimental import pallas as pl
from jax.experimental.pallas import tpu as pltpu
from jax.experimental.pallas import tpu_sc as plsc

_NC = 2
_NS = 16
_K = 125
_CPP = 40
_ZR = 80
_WROWS = 128
_NPAD = 10240


def _mm_body(x_ref, w_ref, b_ref, o_ref):
    o_ref[...] = lax.dot_general(
        x_ref[...], w_ref[...],
        dimension_numbers=(((1,), (1,)), ((), ())),
        preferred_element_type=jnp.float32) + b_ref[...]


def _linear(x, w, b):
    n, d_in = x.shape
    d_out = w.shape[0]
    blk = 1000
    return pl.pallas_call(
        _mm_body,
        grid=(n // blk,),
        in_specs=[
            pl.BlockSpec((blk, d_in), lambda i: (i, 0)),
            pl.BlockSpec((d_out, d_in), lambda i: (0, 0)),
            pl.BlockSpec((1, d_out), lambda i: (0, 0)),
        ],
        out_specs=pl.BlockSpec((blk, d_out), lambda i: (i, 0)),
        out_shape=jax.ShapeDtypeStruct((n, d_out), jnp.float32),
    )(x, w, b.reshape(1, d_out))


def _sc_body(h_hbm, ei_hbm, zero_hbm, p_hbm,
             acc, rowv, colv, gbuf0, gbuf1, sem0, sem1):
    c = lax.axis_index("c")
    s = lax.axis_index("s")
    wid = c * _NS + s
    n_rows = acc.shape[0]
    slab = n_rows // _NS
    chunks = rowv.shape[0]
    npass = ei_hbm.shape[2] // chunks

    zbuf = gbuf0.at[pl.ds(0, _ZR)]
    pltpu.sync_copy(zero_hbm, zbuf)
    for j in range(slab // _ZR):
        pltpu.sync_copy(zbuf, acc.at[pl.ds(s * slab + j * _ZR, _ZR)])
    plsc.subcore_barrier()

    bufs = (gbuf0, gbuf1)
    sems = (sem0, sem1)
    for p in range(npass):
        pltpu.sync_copy(ei_hbm.at[0, wid, pl.ds(p * chunks, chunks)], rowv)
        pltpu.sync_copy(ei_hbm.at[1, wid, pl.ds(p * chunks, chunks)], colv)

        def pair(io, carry):
            i = io * 2
            d0 = pltpu.async_copy(h_hbm.at[colv.at[i]], gbuf0, sem0)
            d1 = pltpu.async_copy(h_hbm.at[colv.at[i + 1]], gbuf1, sem1)
            d0.wait()
            pltpu.sync_copy(gbuf0, acc.at[rowv.at[i]], add=True)
            d1.wait()
            pltpu.sync_copy(gbuf1, acc.at[rowv.at[i + 1]], add=True)
            return carry
        lax.fori_loop(0, chunks // 2, pair, 0)
    plsc.subcore_barrier()

    for j in range(slab // _ZR):
        r0 = s * slab + j * _ZR
        pltpu.sync_copy(acc.at[pl.ds(r0, _ZR)], zbuf)
        pltpu.sync_copy(zbuf, p_hbm.at[pl.ds(c * n_rows + r0, _ZR)])


def _aggregate(h, edge_index):
    n, d = h.shape
    e = edge_index.shape[1]
    nw = _NC * _NS
    npass = -(-e // (_K * nw * _CPP))
    cpw = npass * _CPP
    pad = cpw * _K * nw - e
    if pad:
        trash = n + jnp.arange(pad, dtype=jnp.int32) % (_NPAD - n)
        pad_rc = jnp.stack([trash, jnp.zeros((pad,), jnp.int32)])
        edge_index = jnp.concatenate([edge_index, pad_rc], axis=1)
    ei = edge_index.reshape(2, nw, cpw, _K)
    zeros = jnp.zeros((_ZR, d), jnp.float32)
    mesh = plsc.VectorSubcoreMesh(core_axis_name="c", subcore_axis_name="s",
                                  num_cores=_NC, num_subcores=_NS)
    f = pl.kernel(
        _sc_body,
        out_type=jax.ShapeDtypeStruct((_NC * _NPAD, d), jnp.float32),
        mesh=mesh,
        scratch_types=[
            pltpu.VMEM_SHARED((_NPAD, d), jnp.float32),
            pltpu.VMEM((_CPP, _K), jnp.int32),
            pltpu.VMEM((_CPP, _K), jnp.int32),
            pltpu.VMEM((_K, d), jnp.float32),
            pltpu.VMEM((_K, d), jnp.float32),
            pltpu.SemaphoreType.DMA,
            pltpu.SemaphoreType.DMA,
        ],
    )
    return f(h, ei, zeros)


def _add_body(a_ref, b_ref, o_ref):
    o_ref[...] = a_ref[...] + b_ref[...]


def _combine(p, n, d):
    blk = 1280
    g = -(-n // blk)
    off = _NPAD // blk
    return pl.pallas_call(
        _add_body,
        grid=(g,),
        in_specs=[pl.BlockSpec((blk, d), lambda i: (i, 0)),
                  pl.BlockSpec((blk, d), lambda i: (i + off, 0))],
        out_specs=pl.BlockSpec((blk, d), lambda i: (i, 0)),
        out_shape=jax.ShapeDtypeStruct((n, d), jnp.float32),
    )(p, p)


def kernel(x, edge_index, W, b):
    h = _linear(x, W, b)
    p = _aggregate(h, edge_index)
    n, d = h.shape
    return _combine(p, n, d)
